# Initial kernel scaffold; baseline (speedup 1.0000x reference)
import jax
import jax.numpy as jnp
from jax.experimental import pallas as pl


def kernel(atom_features, pair_features, atom_to_pair, pair_split, edge_index, W_aa, b_aa, W_pp, b_pp, W_ap, b_ap, W_pa, b_pa, W_atom, b_atom, W_pair, b_pair):
    raise NotImplementedError("write your pallas kernel here")


# speedup vs baseline: 2.5252x; 2.5252x over previous
import functools

import jax
import jax.numpy as jnp
from jax import lax
from jax.experimental import pallas as pl
from jax.experimental.pallas import tpu as pltpu
from jax.experimental.pallas import tpu_sc as plsc

N_NODES = 10000
N_EDGES = 320000
D_ATOM = 128
D_PAIR = 16
H = 128

NC = 2
NS = 16
CHUNK = 128
CPT = 160
E_PAD = NS * CPT * CHUNK
ACC_ROWS = 10112
DUMMY = N_NODES
INIT_ROWS = ACC_ROWS // NS
IGRP = 8

@functools.cache
def _mesh():
    return plsc.VectorSubcoreMesh(core_axis_name="c", subcore_axis_name="s",
                                  num_cores=NC, num_subcores=NS)


def _sc_segsum_body(atom_hbm, pa_hbm, src3d, dst3d, ps3d, zeros_hbm,
                    h_out, a1_out, gidx_g, didx_g, rows_v, acc, sem):
    c = lax.axis_index("c")
    s = lax.axis_index("s")
    pltpu.sync_copy(zeros_hbm.at[pl.ds(s * INIT_ROWS, INIT_ROWS)],
                    acc.at[pl.ds(s * INIT_ROWS, INIT_ROWS)])
    plsc.subcore_barrier()

    def group(g, carry):
        @pl.when(c == 0)
        def _():
            pltpu.sync_copy(src3d.at[s, pl.ds(g * IGRP, IGRP)], gidx_g)
            pltpu.sync_copy(dst3d.at[s, pl.ds(g * IGRP, IGRP)], didx_g)

        @pl.when(c == 1)
        def _():
            pltpu.sync_copy(ps3d.at[s, pl.ds(g * IGRP, IGRP)], didx_g)

        def chunk(k, carry2):
            @pl.when(c == 0)
            def _():
                pltpu.async_copy(atom_hbm.at[gidx_g.at[k]], rows_v,
                                 sem).wait()

            @pl.when(c == 1)
            def _():
                base = ((s * CPT + g * IGRP + k) * CHUNK)
                pltpu.sync_copy(pa_hbm.at[pl.ds(base, CHUNK)], rows_v)

            pltpu.sync_copy(rows_v, acc.at[didx_g.at[k]], add=True)
            return carry2

        lax.fori_loop(0, IGRP, chunk, 0)
        return carry

    lax.fori_loop(0, CPT // IGRP, group, 0)
    plsc.subcore_barrier()

    @pl.when(c == 0)
    def _():
        pltpu.sync_copy(acc.at[pl.ds(s * INIT_ROWS, INIT_ROWS)],
                        h_out.at[pl.ds(s * INIT_ROWS, INIT_ROWS)])

    @pl.when(c == 1)
    def _():
        pltpu.sync_copy(acc.at[pl.ds(s * INIT_ROWS, INIT_ROWS)],
                        a1_out.at[pl.ds(s * INIT_ROWS, INIT_ROWS)])


@functools.cache
def _sc_segsum():
    return pl.kernel(
        _sc_segsum_body,
        out_type=[jax.ShapeDtypeStruct((ACC_ROWS, D_ATOM), jnp.float32),
                  jax.ShapeDtypeStruct((ACC_ROWS, H), jnp.float32)],
        mesh=_mesh(),
        scratch_types=[pltpu.VMEM((IGRP, CHUNK), jnp.int32),
                       pltpu.VMEM((IGRP, CHUNK), jnp.int32),
                       pltpu.VMEM((CHUNK, D_ATOM), jnp.float32),
                       pltpu.VMEM_SHARED((ACC_ROWS, D_ATOM), jnp.float32),
                       pltpu.SemaphoreType.DMA],
    )


def _sc_gather_body(h_hbm, i3d, j3d, hgi_out, hgj_out, idx_all, rows_v, sem):
    c = lax.axis_index("c")
    s = lax.axis_index("s")

    @pl.when(c == 0)
    def _():
        pltpu.sync_copy(i3d.at[s], idx_all)

    @pl.when(c == 1)
    def _():
        pltpu.sync_copy(j3d.at[s], idx_all)

    def chunk(k, carry):
        base = (s * CPT + k) * CHUNK
        pltpu.async_copy(h_hbm.at[idx_all.at[k]], rows_v, sem).wait()

        @pl.when(c == 0)
        def _():
            pltpu.sync_copy(rows_v, hgi_out.at[pl.ds(base, CHUNK)])

        @pl.when(c == 1)
        def _():
            pltpu.sync_copy(rows_v, hgj_out.at[pl.ds(base, CHUNK)])

        return carry

    lax.fori_loop(0, CPT, chunk, 0)


@functools.cache
def _sc_gather():
    return pl.kernel(
        _sc_gather_body,
        out_type=[jax.ShapeDtypeStruct((E_PAD, D_ATOM), jnp.float32),
                  jax.ShapeDtypeStruct((E_PAD, D_ATOM), jnp.float32)],
        mesh=_mesh(),
        scratch_types=[pltpu.VMEM((CPT, CHUNK), jnp.int32),
                       pltpu.VMEM((CHUNK, D_ATOM), jnp.float32),
                       pltpu.SemaphoreType.DMA],
    )


def _pa_body(pf_ref, wt_ref, b_ref, out_ref):
    x = jnp.dot(pf_ref[...], wt_ref[...], preferred_element_type=jnp.float32)
    out_ref[...] = jnp.maximum(x + b_ref[...], 0.0)


_PA_BLK = 512


def _tc_pa(pf_pad, WpaT, bpa):
    return pl.pallas_call(
        _pa_body,
        grid=(E_PAD // _PA_BLK,),
        in_specs=[pl.BlockSpec((_PA_BLK, D_PAIR), lambda i: (i, 0)),
                  pl.BlockSpec((D_PAIR, H), lambda i: (0, 0)),
                  pl.BlockSpec((1, H), lambda i: (0, 0))],
        out_specs=pl.BlockSpec((_PA_BLK, H), lambda i: (i, 0)),
        out_shape=jax.ShapeDtypeStruct((E_PAD, H), jnp.float32),
    )(pf_pad, WpaT, bpa)


def _atom_body(h_ref, a1_ref, waaT, baa, wa0T, wa1T, batom, out_ref):
    a0 = jnp.maximum(
        jnp.dot(h_ref[...], waaT[...], preferred_element_type=jnp.float32)
        + baa[...], 0.0)
    x = (jnp.dot(a0, wa0T[...], preferred_element_type=jnp.float32)
         + jnp.dot(a1_ref[...], wa1T[...], preferred_element_type=jnp.float32)
         + batom[...])
    out_ref[...] = jnp.maximum(x, 0.0)


_ATOM_BLK = 1000


def _tc_atom(h, a1, WaaT, baa, Wa0T, Wa1T, batom):
    full = lambda r, cdim: pl.BlockSpec((r, cdim), lambda i: (0, 0))
    return pl.pallas_call(
        _atom_body,
        grid=(N_NODES // _ATOM_BLK,),
        in_specs=[pl.BlockSpec((_ATOM_BLK, D_ATOM), lambda i: (i, 0)),
                  pl.BlockSpec((_ATOM_BLK, H), lambda i: (i, 0)),
                  full(D_ATOM, H), full(1, H), full(H, H), full(H, H),
                  full(1, H)],
        out_specs=pl.BlockSpec((_ATOM_BLK, H), lambda i: (i, 0)),
        out_shape=jax.ShapeDtypeStruct((N_NODES, H), jnp.float32),
    )(h, a1, WaaT, baa, Wa0T, Wa1T, batom)


def _pair_body(hgi_ref, hgj_ref, pf_ref, w1T, w2T, bap, wppT, bpp,
               wp0T, wp1T, bpair, out_ref):
    hgi = hgi_ref[...]
    hgj = hgj_ref[...]
    t1i = jnp.dot(hgi, w1T[...], preferred_element_type=jnp.float32)
    t2j = jnp.dot(hgj, w2T[...], preferred_element_type=jnp.float32)
    t1j = jnp.dot(hgj, w1T[...], preferred_element_type=jnp.float32)
    t2i = jnp.dot(hgi, w2T[...], preferred_element_type=jnp.float32)
    p0 = (jnp.maximum(t1i + t2j + bap[...], 0.0)
          + jnp.maximum(t1j + t2i + bap[...], 0.0))
    p1 = jnp.maximum(
        jnp.dot(pf_ref[...], wppT[...], preferred_element_type=jnp.float32)
        + bpp[...], 0.0)
    x = (jnp.dot(p0, wp0T[...], preferred_element_type=jnp.float32)
         + jnp.dot(p1, wp1T[...], preferred_element_type=jnp.float32)
         + bpair[...])
    out_ref[...] = jnp.maximum(x, 0.0)


_PAIR_BLK = 512


def _tc_pair(hgi, hgj, pf, W1T, W2T, bap, WppT, bpp, Wp0T, Wp1T, bpair):
    full = lambda r, cdim: pl.BlockSpec((r, cdim), lambda i: (0, 0))
    return pl.pallas_call(
        _pair_body,
        grid=(N_EDGES // _PAIR_BLK,),
        in_specs=[pl.BlockSpec((_PAIR_BLK, D_ATOM), lambda i: (i, 0)),
                  pl.BlockSpec((_PAIR_BLK, D_ATOM), lambda i: (i, 0)),
                  pl.BlockSpec((_PAIR_BLK, D_PAIR), lambda i: (i, 0)),
                  full(D_ATOM, H), full(D_ATOM, H), full(1, H),
                  full(D_PAIR, H), full(1, H),
                  full(H, H), full(H, H), full(1, H)],
        out_specs=pl.BlockSpec((_PAIR_BLK, H), lambda i: (i, 0)),
        out_shape=jax.ShapeDtypeStruct((N_EDGES, H), jnp.float32),
    )(hgi, hgj, pf, W1T, W2T, bap, WppT, bpp, Wp0T, Wp1T, bpair)


def _pad_idx(x, value):
    return jnp.concatenate(
        [x, jnp.full((E_PAD - N_EDGES,), value, jnp.int32)]
    ).reshape(NS, CPT, CHUNK)


def kernel(atom_features, pair_features, atom_to_pair, pair_split, edge_index,
           W_aa, b_aa, W_pp, b_pp, W_ap, b_ap, W_pa, b_pa,
           W_atom, b_atom, W_pair, b_pair):
    src3d = _pad_idx(edge_index[0], 0)
    dst3d = _pad_idx(edge_index[1], DUMMY)
    ps3d = _pad_idx(pair_split, DUMMY)
    i3d = _pad_idx(atom_to_pair[:, 0], 0)
    j3d = _pad_idx(atom_to_pair[:, 1], 0)
    pf_pad = jnp.concatenate(
        [pair_features, jnp.zeros((E_PAD - N_EDGES, D_PAIR), jnp.float32)])
    zeros = jnp.zeros((ACC_ROWS, D_ATOM), jnp.float32)

    pa = _tc_pa(pf_pad, W_pa.T, b_pa[None])
    h, a1 = _sc_segsum()(atom_features, pa, src3d, dst3d, ps3d, zeros)
    hgi, hgj = _sc_gather()(h, i3d, j3d)

    next_atom = _tc_atom(h, a1, W_aa.T, b_aa[None],
                         W_atom[:, :H].T, W_atom[:, H:].T, b_atom[None])
    next_pair = _tc_pair(hgi, hgj, pair_features,
                         W_ap[:, :D_ATOM].T, W_ap[:, D_ATOM:].T, b_ap[None],
                         W_pp.T, b_pp[None],
                         W_pair[:, :H].T, W_pair[:, H:].T, b_pair[None])
    return (next_atom, next_pair)


# speedup vs baseline: 2.7929x; 1.1060x over previous
import functools

import jax
import jax.numpy as jnp
from jax import lax
from jax.experimental import pallas as pl
from jax.experimental.pallas import tpu as pltpu
from jax.experimental.pallas import tpu_sc as plsc

N_NODES = 10000
N_EDGES = 320000
D_ATOM = 128
D_PAIR = 16
H = 128

NC = 2
NS = 16
CHUNK = 128
CPT = 160
E_PAD = NS * CPT * CHUNK
ACC_ROWS = 10112
DUMMY = N_NODES
INIT_ROWS = ACC_ROWS // NS
IGRP = 16

@functools.cache
def _mesh():
    return plsc.VectorSubcoreMesh(core_axis_name="c", subcore_axis_name="s",
                                  num_cores=NC, num_subcores=NS)


def _sc_segsum_body(atom_hbm, pa_hbm, src3d, dst3d, ps3d, zeros_hbm,
                    h_out, a1_out, gidx_g, didx_g, rows0, rows1, acc,
                    sem0, sem1):
    c = lax.axis_index("c")
    s = lax.axis_index("s")
    pltpu.sync_copy(zeros_hbm.at[pl.ds(s * INIT_ROWS, INIT_ROWS)],
                    acc.at[pl.ds(s * INIT_ROWS, INIT_ROWS)])
    plsc.subcore_barrier()

    def start(g, k, buf, sem):
        @pl.when(c == 0)
        def _():
            pltpu.async_copy(atom_hbm.at[gidx_g.at[k]], buf, sem)

        @pl.when(c == 1)
        def _():
            base = (s * CPT + g * IGRP + k) * CHUNK
            pltpu.async_copy(pa_hbm.at[pl.ds(base, CHUNK)], buf, sem)

    def wait(buf, sem):
        pltpu.make_async_copy(atom_hbm.at[pl.ds(0, CHUNK)], buf, sem).wait()

    def group(g, carry):
        @pl.when(c == 0)
        def _():
            pltpu.sync_copy(src3d.at[s, pl.ds(g * IGRP, IGRP)], gidx_g)
            pltpu.sync_copy(dst3d.at[s, pl.ds(g * IGRP, IGRP)], didx_g)

        @pl.when(c == 1)
        def _():
            pltpu.sync_copy(ps3d.at[s, pl.ds(g * IGRP, IGRP)], didx_g)

        start(g, 0, rows0, sem0)

        def pair(k2, carry2):
            k = 2 * k2
            start(g, k + 1, rows1, sem1)
            wait(rows0, sem0)
            pltpu.sync_copy(rows0, acc.at[didx_g.at[k]], add=True)

            @pl.when(k + 2 < IGRP)
            def _():
                start(g, k + 2, rows0, sem0)

            wait(rows1, sem1)
            pltpu.sync_copy(rows1, acc.at[didx_g.at[k + 1]], add=True)
            return carry2

        lax.fori_loop(0, IGRP // 2, pair, 0)
        return carry

    lax.fori_loop(0, CPT // IGRP, group, 0)
    plsc.subcore_barrier()

    @pl.when(c == 0)
    def _():
        pltpu.sync_copy(acc.at[pl.ds(s * INIT_ROWS, INIT_ROWS)],
                        h_out.at[pl.ds(s * INIT_ROWS, INIT_ROWS)])

    @pl.when(c == 1)
    def _():
        pltpu.sync_copy(acc.at[pl.ds(s * INIT_ROWS, INIT_ROWS)],
                        a1_out.at[pl.ds(s * INIT_ROWS, INIT_ROWS)])


@functools.cache
def _sc_segsum():
    return pl.kernel(
        _sc_segsum_body,
        out_type=[jax.ShapeDtypeStruct((ACC_ROWS, D_ATOM), jnp.float32),
                  jax.ShapeDtypeStruct((ACC_ROWS, H), jnp.float32)],
        mesh=_mesh(),
        scratch_types=[pltpu.VMEM((IGRP, CHUNK), jnp.int32),
                       pltpu.VMEM((IGRP, CHUNK), jnp.int32),
                       pltpu.VMEM((CHUNK, D_ATOM), jnp.float32),
                       pltpu.VMEM((CHUNK, D_ATOM), jnp.float32),
                       pltpu.VMEM_SHARED((ACC_ROWS, D_ATOM), jnp.float32),
                       pltpu.SemaphoreType.DMA,
                       pltpu.SemaphoreType.DMA],
    )


def _sc_gather_body(h_hbm, i3d, j3d, hgi_out, hgj_out, idx_all, rows0, rows1,
                    sem0, sem1):
    c = lax.axis_index("c")
    s = lax.axis_index("s")

    @pl.when(c == 0)
    def _():
        pltpu.sync_copy(i3d.at[s], idx_all)

    @pl.when(c == 1)
    def _():
        pltpu.sync_copy(j3d.at[s], idx_all)

    def wait(buf, sem):
        pltpu.make_async_copy(h_hbm.at[pl.ds(0, CHUNK)], buf, sem).wait()

    def write(k, buf):
        base = (s * CPT + k) * CHUNK

        @pl.when(c == 0)
        def _():
            pltpu.sync_copy(buf, hgi_out.at[pl.ds(base, CHUNK)])

        @pl.when(c == 1)
        def _():
            pltpu.sync_copy(buf, hgj_out.at[pl.ds(base, CHUNK)])

    pltpu.async_copy(h_hbm.at[idx_all.at[0]], rows0, sem0)

    def pair(k2, carry):
        k = 2 * k2
        pltpu.async_copy(h_hbm.at[idx_all.at[k + 1]], rows1, sem1)
        wait(rows0, sem0)
        write(k, rows0)

        @pl.when(k + 2 < CPT)
        def _():
            pltpu.async_copy(h_hbm.at[idx_all.at[k + 2]], rows0, sem0)

        wait(rows1, sem1)
        write(k + 1, rows1)
        return carry

    lax.fori_loop(0, CPT // 2, pair, 0)


@functools.cache
def _sc_gather():
    return pl.kernel(
        _sc_gather_body,
        out_type=[jax.ShapeDtypeStruct((E_PAD, D_ATOM), jnp.float32),
                  jax.ShapeDtypeStruct((E_PAD, D_ATOM), jnp.float32)],
        mesh=_mesh(),
        scratch_types=[pltpu.VMEM((CPT, CHUNK), jnp.int32),
                       pltpu.VMEM((CHUNK, D_ATOM), jnp.float32),
                       pltpu.VMEM((CHUNK, D_ATOM), jnp.float32),
                       pltpu.SemaphoreType.DMA,
                       pltpu.SemaphoreType.DMA],
    )


def _pa_body(pf_ref, wt_ref, b_ref, out_ref):
    x = jnp.dot(pf_ref[...], wt_ref[...], preferred_element_type=jnp.float32)
    out_ref[...] = jnp.maximum(x + b_ref[...], 0.0)


_PA_BLK = 512
_PA_PACK = 8


def _tc_pa(pf_pad, WpaT, bpa):
    pf8 = pf_pad.reshape(E_PAD // _PA_PACK, _PA_PACK * D_PAIR)
    wbig = (jnp.eye(_PA_PACK, dtype=jnp.float32)[:, None, :, None]
            * WpaT[None, :, None, :]).reshape(_PA_PACK * D_PAIR,
                                              _PA_PACK * H)
    b8 = jnp.tile(bpa, (1, _PA_PACK))
    out = pl.pallas_call(
        _pa_body,
        grid=(E_PAD // _PA_PACK // _PA_BLK,),
        in_specs=[pl.BlockSpec((_PA_BLK, _PA_PACK * D_PAIR),
                               lambda i: (i, 0)),
                  pl.BlockSpec((_PA_PACK * D_PAIR, _PA_PACK * H),
                               lambda i: (0, 0)),
                  pl.BlockSpec((1, _PA_PACK * H), lambda i: (0, 0))],
        out_specs=pl.BlockSpec((_PA_BLK, _PA_PACK * H), lambda i: (i, 0)),
        out_shape=jax.ShapeDtypeStruct((E_PAD // _PA_PACK, _PA_PACK * H),
                                       jnp.float32),
    )(pf8, wbig, b8)
    return out.reshape(E_PAD, H)


def _atom_body(h_ref, a1_ref, waaT, baa, wa0T, wa1T, batom, out_ref):
    a0 = jnp.maximum(
        jnp.dot(h_ref[...], waaT[...], preferred_element_type=jnp.float32)
        + baa[...], 0.0)
    x = (jnp.dot(a0, wa0T[...], preferred_element_type=jnp.float32)
         + jnp.dot(a1_ref[...], wa1T[...], preferred_element_type=jnp.float32)
         + batom[...])
    out_ref[...] = jnp.maximum(x, 0.0)


_ATOM_BLK = 1000


def _tc_atom(h, a1, WaaT, baa, Wa0T, Wa1T, batom):
    full = lambda r, cdim: pl.BlockSpec((r, cdim), lambda i: (0, 0))
    return pl.pallas_call(
        _atom_body,
        grid=(N_NODES // _ATOM_BLK,),
        in_specs=[pl.BlockSpec((_ATOM_BLK, D_ATOM), lambda i: (i, 0)),
                  pl.BlockSpec((_ATOM_BLK, H), lambda i: (i, 0)),
                  full(D_ATOM, H), full(1, H), full(H, H), full(H, H),
                  full(1, H)],
        out_specs=pl.BlockSpec((_ATOM_BLK, H), lambda i: (i, 0)),
        out_shape=jax.ShapeDtypeStruct((N_NODES, H), jnp.float32),
    )(h, a1, WaaT, baa, Wa0T, Wa1T, batom)


def _pair_body(hgi_ref, hgj_ref, pf_ref, w1T, w2T, bap, wppT, bpp,
               wp0T, wp1T, bpair, out_ref):
    hgi = hgi_ref[...]
    hgj = hgj_ref[...]
    t1i = jnp.dot(hgi, w1T[...], preferred_element_type=jnp.float32)
    t2j = jnp.dot(hgj, w2T[...], preferred_element_type=jnp.float32)
    t1j = jnp.dot(hgj, w1T[...], preferred_element_type=jnp.float32)
    t2i = jnp.dot(hgi, w2T[...], preferred_element_type=jnp.float32)
    p0 = (jnp.maximum(t1i + t2j + bap[...], 0.0)
          + jnp.maximum(t1j + t2i + bap[...], 0.0))
    p1 = jnp.maximum(
        jnp.dot(pf_ref[...], wppT[...], preferred_element_type=jnp.float32)
        + bpp[...], 0.0)
    x = (jnp.dot(p0, wp0T[...], preferred_element_type=jnp.float32)
         + jnp.dot(p1, wp1T[...], preferred_element_type=jnp.float32)
         + bpair[...])
    out_ref[...] = jnp.maximum(x, 0.0)


_PAIR_BLK = 512


def _tc_pair(hgi, hgj, pf, W1T, W2T, bap, WppT, bpp, Wp0T, Wp1T, bpair):
    full = lambda r, cdim: pl.BlockSpec((r, cdim), lambda i: (0, 0))
    return pl.pallas_call(
        _pair_body,
        grid=(N_EDGES // _PAIR_BLK,),
        in_specs=[pl.BlockSpec((_PAIR_BLK, D_ATOM), lambda i: (i, 0)),
                  pl.BlockSpec((_PAIR_BLK, D_ATOM), lambda i: (i, 0)),
                  pl.BlockSpec((_PAIR_BLK, D_PAIR), lambda i: (i, 0)),
                  full(D_ATOM, H), full(D_ATOM, H), full(1, H),
                  full(D_PAIR, H), full(1, H),
                  full(H, H), full(H, H), full(1, H)],
        out_specs=pl.BlockSpec((_PAIR_BLK, H), lambda i: (i, 0)),
        out_shape=jax.ShapeDtypeStruct((N_EDGES, H), jnp.float32),
    )(hgi, hgj, pf, W1T, W2T, bap, WppT, bpp, Wp0T, Wp1T, bpair)


def _pad_idx(x, value):
    return jnp.concatenate(
        [x, jnp.full((E_PAD - N_EDGES,), value, jnp.int32)]
    ).reshape(NS, CPT, CHUNK)


def kernel(atom_features, pair_features, atom_to_pair, pair_split, edge_index,
           W_aa, b_aa, W_pp, b_pp, W_ap, b_ap, W_pa, b_pa,
           W_atom, b_atom, W_pair, b_pair):
    src3d = _pad_idx(edge_index[0], 0)
    dst3d = _pad_idx(edge_index[1], DUMMY)
    ps3d = _pad_idx(pair_split, DUMMY)
    i3d = _pad_idx(atom_to_pair[:, 0], 0)
    j3d = _pad_idx(atom_to_pair[:, 1], 0)
    pf_pad = jnp.concatenate(
        [pair_features, jnp.zeros((E_PAD - N_EDGES, D_PAIR), jnp.float32)])
    zeros = jnp.zeros((ACC_ROWS, D_ATOM), jnp.float32)

    pa = _tc_pa(pf_pad, W_pa.T, b_pa[None])
    h, a1 = _sc_segsum()(atom_features, pa, src3d, dst3d, ps3d, zeros)
    hgi, hgj = _sc_gather()(h, i3d, j3d)

    next_atom = _tc_atom(h, a1, W_aa.T, b_aa[None],
                         W_atom[:, :H].T, W_atom[:, H:].T, b_atom[None])
    next_pair = _tc_pair(hgi, hgj, pair_features,
                         W_ap[:, :D_ATOM].T, W_ap[:, D_ATOM:].T, b_ap[None],
                         W_pp.T, b_pp[None],
                         W_pair[:, :H].T, W_pair[:, H:].T, b_pair[None])
    return (next_atom, next_pair)


# speedup vs baseline: 3.0649x; 1.0974x over previous
import functools

import jax
import jax.numpy as jnp
from jax import lax
from jax.experimental import pallas as pl
from jax.experimental.pallas import tpu as pltpu
from jax.experimental.pallas import tpu_sc as plsc

N_NODES = 10000
N_EDGES = 320000
D_ATOM = 128
D_PAIR = 16
H = 128

NC = 2
NS = 16
CHUNK = 128
NROWS = 2560
E_PAD = NROWS * CHUNK
CPT_W = NROWS // (NC * NS)
CPT_C = NROWS // NS
ACC_ROWS = 10112
DUMMY = N_NODES
INIT_ROWS = ACC_ROWS // NS
IGRP = 16
PACK = 8


@functools.cache
def _mesh():
    return plsc.VectorSubcoreMesh(core_axis_name="c", subcore_axis_name="s",
                                  num_cores=NC, num_subcores=NS)


def _wait(dummy_ref, buf, sem):
    pltpu.make_async_copy(dummy_ref.at[pl.ds(0, CHUNK)], buf, sem).wait()


def _sc_h_body(atom_hbm, src2d, dst2d, zeros_hbm, hpart,
               gidx_g, didx_g, rows0, rows1, acc, sem0, sem1):
    c = lax.axis_index("c")
    s = lax.axis_index("s")
    w = c * NS + s
    pltpu.sync_copy(zeros_hbm.at[pl.ds(s * INIT_ROWS, INIT_ROWS)],
                    acc.at[pl.ds(s * INIT_ROWS, INIT_ROWS)])
    plsc.subcore_barrier()

    def start(k, buf, sem):
        pltpu.async_copy(atom_hbm.at[gidx_g.at[k]], buf, sem)

    def group(g, carry):
        base = w * CPT_W + g * IGRP
        pltpu.sync_copy(src2d.at[pl.ds(base, IGRP)], gidx_g)
        pltpu.sync_copy(dst2d.at[pl.ds(base, IGRP)], didx_g)
        start(0, rows0, sem0)

        def pair(k2, carry2):
            k = 2 * k2
            start(k + 1, rows1, sem1)
            _wait(atom_hbm, rows0, sem0)
            pltpu.sync_copy(rows0, acc.at[didx_g.at[k]], add=True)

            @pl.when(k + 2 < IGRP)
            def _():
                start(k + 2, rows0, sem0)

            _wait(atom_hbm, rows1, sem1)
            pltpu.sync_copy(rows1, acc.at[didx_g.at[k + 1]], add=True)
            return carry2

        lax.fori_loop(0, IGRP // 2, pair, 0)
        return carry

    lax.fori_loop(0, CPT_W // IGRP, group, 0)
    plsc.subcore_barrier()
    pltpu.sync_copy(acc.at[pl.ds(s * INIT_ROWS, INIT_ROWS)],
                    hpart.at[c, pl.ds(s * INIT_ROWS, INIT_ROWS)])


@functools.cache
def _sc_h():
    return pl.kernel(
        _sc_h_body,
        out_type=jax.ShapeDtypeStruct((NC, ACC_ROWS, D_ATOM), jnp.float32),
        mesh=_mesh(),
        scratch_types=[pltpu.VMEM((IGRP, CHUNK), jnp.int32),
                       pltpu.VMEM((IGRP, CHUNK), jnp.int32),
                       pltpu.VMEM((CHUNK, D_ATOM), jnp.float32),
                       pltpu.VMEM((CHUNK, D_ATOM), jnp.float32),
                       pltpu.VMEM_SHARED((ACC_ROWS, D_ATOM), jnp.float32),
                       pltpu.SemaphoreType.DMA,
                       pltpu.SemaphoreType.DMA],
    )


def _sc_gather_body(h_hbm, i2d, j2d, hgi_out, hgj_out,
                    idx_all, rows0, rows1, sem0, sem1):
    c = lax.axis_index("c")
    s = lax.axis_index("s")

    @pl.when(c == 0)
    def _():
        pltpu.sync_copy(i2d.at[pl.ds(s * CPT_C, CPT_C)], idx_all)

    @pl.when(c == 1)
    def _():
        pltpu.sync_copy(j2d.at[pl.ds(s * CPT_C, CPT_C)], idx_all)

    def write(k, buf):
        base = (s * CPT_C + k) * CHUNK

        @pl.when(c == 0)
        def _():
            pltpu.sync_copy(buf, hgi_out.at[pl.ds(base, CHUNK)])

        @pl.when(c == 1)
        def _():
            pltpu.sync_copy(buf, hgj_out.at[pl.ds(base, CHUNK)])

    pltpu.async_copy(h_hbm.at[idx_all.at[0]], rows0, sem0)

    def pair(k2, carry):
        k = 2 * k2
        pltpu.async_copy(h_hbm.at[idx_all.at[k + 1]], rows1, sem1)
        _wait(h_hbm, rows0, sem0)
        write(k, rows0)

        @pl.when(k + 2 < CPT_C)
        def _():
            pltpu.async_copy(h_hbm.at[idx_all.at[k + 2]], rows0, sem0)

        _wait(h_hbm, rows1, sem1)
        write(k + 1, rows1)
        return carry

    lax.fori_loop(0, CPT_C // 2, pair, 0)


@functools.cache
def _sc_gather():
    return pl.kernel(
        _sc_gather_body,
        out_type=[jax.ShapeDtypeStruct((E_PAD, D_ATOM), jnp.float32),
                  jax.ShapeDtypeStruct((E_PAD, D_ATOM), jnp.float32)],
        mesh=_mesh(),
        scratch_types=[pltpu.VMEM((CPT_C, CHUNK), jnp.int32),
                       pltpu.VMEM((CHUNK, D_ATOM), jnp.float32),
                       pltpu.VMEM((CHUNK, D_ATOM), jnp.float32),
                       pltpu.SemaphoreType.DMA,
                       pltpu.SemaphoreType.DMA],
    )


def _sc_a1_body(pa_hbm, ps2d, zeros_hbm, a1part,
                didx_g, rows0, rows1, acc, sem0, sem1):
    c = lax.axis_index("c")
    s = lax.axis_index("s")
    w = c * NS + s
    pltpu.sync_copy(zeros_hbm.at[pl.ds(s * INIT_ROWS, INIT_ROWS)],
                    acc.at[pl.ds(s * INIT_ROWS, INIT_ROWS)])
    plsc.subcore_barrier()

    def start(base_row, k, buf, sem):
        pltpu.async_copy(pa_hbm.at[pl.ds((base_row + k) * CHUNK, CHUNK)],
                         buf, sem)

    def group(g, carry):
        base = w * CPT_W + g * IGRP
        pltpu.sync_copy(ps2d.at[pl.ds(base, IGRP)], didx_g)
        start(base, 0, rows0, sem0)

        def pair(k2, carry2):
            k = 2 * k2
            start(base, k + 1, rows1, sem1)
            _wait(pa_hbm, rows0, sem0)
            pltpu.sync_copy(rows0, acc.at[didx_g.at[k]], add=True)

            @pl.when(k + 2 < IGRP)
            def _():
                start(base, k + 2, rows0, sem0)

            _wait(pa_hbm, rows1, sem1)
            pltpu.sync_copy(rows1, acc.at[didx_g.at[k + 1]], add=True)
            return carry2

        lax.fori_loop(0, IGRP // 2, pair, 0)
        return carry

    lax.fori_loop(0, CPT_W // IGRP, group, 0)
    plsc.subcore_barrier()
    pltpu.sync_copy(acc.at[pl.ds(s * INIT_ROWS, INIT_ROWS)],
                    a1part.at[c, pl.ds(s * INIT_ROWS, INIT_ROWS)])


@functools.cache
def _sc_a1():
    return pl.kernel(
        _sc_a1_body,
        out_type=jax.ShapeDtypeStruct((NC, ACC_ROWS, H), jnp.float32),
        mesh=_mesh(),
        scratch_types=[pltpu.VMEM((IGRP, CHUNK), jnp.int32),
                       pltpu.VMEM((CHUNK, H), jnp.float32),
                       pltpu.VMEM((CHUNK, H), jnp.float32),
                       pltpu.VMEM_SHARED((ACC_ROWS, H), jnp.float32),
                       pltpu.SemaphoreType.DMA,
                       pltpu.SemaphoreType.DMA],
    )


def _pa_body(pf_ref, wt_ref, b_ref, out_ref):
    x = jnp.dot(pf_ref[...], wt_ref[...], preferred_element_type=jnp.float32)
    out_ref[...] = jnp.maximum(x + b_ref[...], 0.0)


_PA_BLK = 512


def _tc_pa(pf8, WpaT, bpa):
    wbig = (jnp.eye(PACK, dtype=jnp.float32)[:, None, :, None]
            * WpaT[None, :, None, :]).reshape(PACK * D_PAIR, PACK * H)
    b8 = jnp.tile(bpa, (1, PACK))
    out = pl.pallas_call(
        _pa_body,
        grid=(E_PAD // PACK // _PA_BLK,),
        in_specs=[pl.BlockSpec((_PA_BLK, PACK * D_PAIR), lambda i: (i, 0)),
                  pl.BlockSpec((PACK * D_PAIR, PACK * H), lambda i: (0, 0)),
                  pl.BlockSpec((1, PACK * H), lambda i: (0, 0))],
        out_specs=pl.BlockSpec((_PA_BLK, PACK * H), lambda i: (i, 0)),
        out_shape=jax.ShapeDtypeStruct((E_PAD // PACK, PACK * H),
                                       jnp.float32),
    )(pf8, wbig, b8)
    return out.reshape(E_PAD, H)


def _hsum_body(hp0_ref, hp1_ref, waaT, baa, h_ref, a0_ref):
    h = hp0_ref[0] + hp1_ref[0]
    h_ref[...] = h
    a0 = jnp.dot(h, waaT[...], preferred_element_type=jnp.float32) + baa[...]
    a0_ref[...] = jnp.maximum(a0, 0.0)


_HS_BLK = 128


def _tc_hsum(hpart, WaaT, baa):
    return pl.pallas_call(
        _hsum_body,
        grid=(ACC_ROWS // _HS_BLK,),
        in_specs=[pl.BlockSpec((1, _HS_BLK, D_ATOM), lambda i: (0, i, 0)),
                  pl.BlockSpec((1, _HS_BLK, D_ATOM), lambda i: (1, i, 0)),
                  pl.BlockSpec((D_ATOM, H), lambda i: (0, 0)),
                  pl.BlockSpec((1, H), lambda i: (0, 0))],
        out_specs=[pl.BlockSpec((_HS_BLK, D_ATOM), lambda i: (i, 0)),
                   pl.BlockSpec((_HS_BLK, H), lambda i: (i, 0))],
        out_shape=[jax.ShapeDtypeStruct((ACC_ROWS, D_ATOM), jnp.float32),
                   jax.ShapeDtypeStruct((ACC_ROWS, H), jnp.float32)],
    )(hpart, hpart, WaaT, baa)


def _atom_body(a0_ref, a1p0_ref, a1p1_ref, wa0T, wa1T, batom, out_ref):
    a1 = a1p0_ref[0] + a1p1_ref[0]
    x = (jnp.dot(a0_ref[...], wa0T[...], preferred_element_type=jnp.float32)
         + jnp.dot(a1, wa1T[...], preferred_element_type=jnp.float32)
         + batom[...])
    out_ref[...] = jnp.maximum(x, 0.0)


_ATOM_BLK = 1000


def _tc_atom(a0, a1part, Wa0T, Wa1T, batom):
    full = lambda r, cdim: pl.BlockSpec((r, cdim), lambda i: (0, 0))
    return pl.pallas_call(
        _atom_body,
        grid=(N_NODES // _ATOM_BLK,),
        in_specs=[pl.BlockSpec((_ATOM_BLK, H), lambda i: (i, 0)),
                  pl.BlockSpec((1, _ATOM_BLK, H), lambda i: (0, i, 0)),
                  pl.BlockSpec((1, _ATOM_BLK, H), lambda i: (1, i, 0)),
                  full(H, H), full(H, H), full(1, H)],
        out_specs=pl.BlockSpec((_ATOM_BLK, H), lambda i: (i, 0)),
        out_shape=jax.ShapeDtypeStruct((N_NODES, H), jnp.float32),
    )(a0, a1part, a1part, Wa0T, Wa1T, batom)


def _pair_body(hgi_ref, hgj_ref, pf8_ref, w1T, w2T, bap, wppbig, bpp8,
               wp0T, wp1T, bpair, out_ref):
    hgi = hgi_ref[...]
    hgj = hgj_ref[...]
    t1i = jnp.dot(hgi, w1T[...], preferred_element_type=jnp.float32)
    t2j = jnp.dot(hgj, w2T[...], preferred_element_type=jnp.float32)
    t1j = jnp.dot(hgj, w1T[...], preferred_element_type=jnp.float32)
    t2i = jnp.dot(hgi, w2T[...], preferred_element_type=jnp.float32)
    p0 = (jnp.maximum(t1i + t2j + bap[...], 0.0)
          + jnp.maximum(t1j + t2i + bap[...], 0.0))
    p18 = jnp.maximum(
        jnp.dot(pf8_ref[...], wppbig[...],
                preferred_element_type=jnp.float32) + bpp8[...], 0.0)
    p1 = p18.reshape(_PAIR_BLK, H)
    x = (jnp.dot(p0, wp0T[...], preferred_element_type=jnp.float32)
         + jnp.dot(p1, wp1T[...], preferred_element_type=jnp.float32)
         + bpair[...])
    out_ref[...] = jnp.maximum(x, 0.0)


_PAIR_BLK = 512


def _tc_pair(hgi, hgj, pf8, W1T, W2T, bap, WppT, bpp, Wp0T, Wp1T, bpair):
    wppbig = (jnp.eye(PACK, dtype=jnp.float32)[:, None, :, None]
              * WppT[None, :, None, :]).reshape(PACK * D_PAIR, PACK * H)
    bpp8 = jnp.tile(bpp, (1, PACK))
    full = lambda r, cdim: pl.BlockSpec((r, cdim), lambda i: (0, 0))
    return pl.pallas_call(
        _pair_body,
        grid=(N_EDGES // _PAIR_BLK,),
        in_specs=[pl.BlockSpec((_PAIR_BLK, D_ATOM), lambda i: (i, 0)),
                  pl.BlockSpec((_PAIR_BLK, D_ATOM), lambda i: (i, 0)),
                  pl.BlockSpec((_PAIR_BLK // PACK, PACK * D_PAIR),
                               lambda i: (i, 0)),
                  full(D_ATOM, H), full(D_ATOM, H), full(1, H),
                  full(PACK * D_PAIR, PACK * H), full(1, PACK * H),
                  full(H, H), full(H, H), full(1, H)],
        out_specs=pl.BlockSpec((_PAIR_BLK, H), lambda i: (i, 0)),
        out_shape=jax.ShapeDtypeStruct((N_EDGES, H), jnp.float32),
    )(hgi, hgj, pf8, W1T, W2T, bap, wppbig, bpp8, Wp0T, Wp1T, bpair)


def _pad_idx(x, value):
    return jnp.concatenate(
        [x, jnp.full((E_PAD - N_EDGES,), value, jnp.int32)]
    ).reshape(NROWS, CHUNK)


def kernel(atom_features, pair_features, atom_to_pair, pair_split, edge_index,
           W_aa, b_aa, W_pp, b_pp, W_ap, b_ap, W_pa, b_pa,
           W_atom, b_atom, W_pair, b_pair):
    src2d = _pad_idx(edge_index[0], 0)
    dst2d = _pad_idx(edge_index[1], DUMMY)
    ps2d = _pad_idx(pair_split, DUMMY)
    i2d = _pad_idx(atom_to_pair[:, 0], 0)
    j2d = _pad_idx(atom_to_pair[:, 1], 0)
    pf8 = jnp.concatenate(
        [pair_features,
         jnp.zeros((E_PAD - N_EDGES, D_PAIR), jnp.float32)]
    ).reshape(E_PAD // PACK, PACK * D_PAIR)
    zeros = jnp.zeros((ACC_ROWS, D_ATOM), jnp.float32)

    hpart = _sc_h()(atom_features, src2d, dst2d, zeros)
    pa = _tc_pa(pf8, W_pa.T, b_pa[None])
    h, a0 = _tc_hsum(hpart, W_aa.T, b_aa[None])
    hgi, hgj = _sc_gather()(h, i2d, j2d)
    a1part = _sc_a1()(pa, ps2d, zeros)

    next_atom = _tc_atom(a0, a1part, W_atom[:, :H].T, W_atom[:, H:].T,
                         b_atom[None])
    next_pair = _tc_pair(hgi, hgj, pf8,
                         W_ap[:, :D_ATOM].T, W_ap[:, D_ATOM:].T, b_ap[None],
                         W_pp.T, b_pp[None],
                         W_pair[:, :H].T, W_pair[:, H:].T, b_pair[None])
    return (next_atom, next_pair)
